# Initial kernel scaffold; baseline (speedup 1.0000x reference)
import functools

import jax
import jax.numpy as jnp
from jax import lax
from jax.experimental import pallas as pl
from jax.experimental.pallas import tpu as pltpu
from jax.experimental.pallas import tpu_sc as plsc

LEVEL = 6
V = 10 * 4 ** LEVEL + 2
F = 20 * 4 ** LEVEL
NVP = 10 * 4 ** (LEVEL - 1) + 2
B = 2
CIN = 64
COUT = 32

LANE = 16
VBLK = 512
NVB = 81
VP = NVB * VBLK
G = VP // LANE
NVR = 4
NCG = 8
GPT = G // NVR
CHUNK = 8
NCHUNK = GPT // CHUNK


def _sc_body(tidx_hbm, tw_hbm, xr_hbm, s_hbm, xpart, tidxv, twv, sbuf):
    wid = lax.axis_index("s") * 2 + lax.axis_index("c")
    cg = wid % NCG
    vr = wid // NCG

    pltpu.sync_copy(xr_hbm.at[cg], xpart)

    cvs = [jnp.full((LANE,), c, dtype=jnp.int32) for c in range(8)]

    def chunk_body(ci, carry):
        g0 = vr * GPT + ci * CHUNK
        pltpu.sync_copy(tidx_hbm.at[pl.ds(g0, CHUNK)], tidxv)
        pltpu.sync_copy(tw_hbm.at[pl.ds(g0, CHUNK)], twv)

        def group_body(gi, carry2):
            acc = [[jnp.zeros((LANE,), jnp.float32) for _ in range(8)]
                   for _ in range(4)]
            for j in range(26):
                idxv = tidxv[gi, j]
                if j == 0:
                    w0 = twv[gi, 0]
                    for c in range(8):
                        g = plsc.load_gather(xpart, [idxv, cvs[c]])
                        acc[0][c] = acc[0][c] + w0 * g
                elif j < 8:
                    wl = twv[gi, j]
                    for c in range(8):
                        g = plsc.load_gather(xpart, [idxv, cvs[c]])
                        acc[1][c] = acc[1][c] + wl * g
                else:
                    we = twv[gi, j]
                    wn = twv[gi, j + 18]
                    for c in range(8):
                        g = plsc.load_gather(xpart, [idxv, cvs[c]])
                        acc[2][c] = acc[2][c] + we * g
                        acc[3][c] = acc[3][c] + wn * g
            for k in range(4):
                for c in range(8):
                    sbuf[k, c, pl.ds(gi * LANE, LANE)] = acc[k][c]
            return carry2

        lax.fori_loop(0, CHUNK, group_body, 0, unroll=False)
        pltpu.sync_copy(
            sbuf, s_hbm.at[:, pl.ds(cg * 8, 8), pl.ds(g0 * LANE, CHUNK * LANE)])
        return carry

    lax.fori_loop(0, NCHUNK, chunk_body, 0, unroll=False)


def _tc_body(s_ref, x2_ref, a4_ref, wc_ref, bias_ref, o_ref):
    acc = bias_ref[...].astype(jnp.float32)
    for k in range(4):
        acc = acc + lax.dot_general(
            a4_ref[k], s_ref[k],
            dimension_numbers=(((1,), (0,)), ((), ())),
            preferred_element_type=jnp.float32)
    acc = acc + lax.dot_general(
        wc_ref[...], x2_ref[0],
        dimension_numbers=(((1,), (0,)), ((), ())),
        preferred_element_type=jnp.float32)
    o_ref[0] = acc


def kernel(x1, x2, coeffs, bias_up, Wc, bc, L_rows, L_cols, L_vals,
           G_rows, G_cols, G_vals, EW, NS, F2V_rows, F2V_cols, F2V_vals):
    f32 = jnp.float32
    i32 = jnp.int32

    Lc = L_cols.reshape(V, 7).astype(i32)
    Lv = L_vals.reshape(V, 7)
    fverts = G_cols.reshape(3 * F, 3)[:F].astype(i32)
    Gv_e = G_vals.reshape(3, F, 3)
    wew = jnp.einsum('fe,eft->ft', EW, Gv_e)
    wns = jnp.einsum('fe,eft->ft', NS, Gv_e)
    fv = F2V_cols.reshape(V, 6).astype(i32)
    fw = F2V_vals.reshape(V, 6)
    gidx = fverts[fv].reshape(V, 18)
    g_ew = (fw[..., None] * wew[fv]).reshape(V, 18)
    g_ns = (fw[..., None] * wns[fv]).reshape(V, 18)
    iidx = jnp.arange(V, dtype=i32)[:, None]
    iw = (iidx < NVP).astype(f32)

    IDX = jnp.concatenate([iidx, Lc, gidx], axis=1)
    ok = IDX < NVP
    IDXm = jnp.where(ok, IDX, 0)
    Wt = jnp.concatenate(
        [iw * ok[:, :1], Lv * ok[:, 1:8],
         g_ew * ok[:, 8:26], g_ns * ok[:, 8:26]], axis=1)

    IDXp = jnp.zeros((VP, 26), i32).at[:V].set(IDXm)
    Wtp = jnp.zeros((VP, 44), f32).at[:V].set(Wt)
    Tidx = IDXp.reshape(G, LANE, 26).transpose(0, 2, 1)
    Tw = Wtp.reshape(G, LANE, 44).transpose(0, 2, 1)

    Xr = x1.transpose(2, 0, 1).reshape(NVP, 8, 8).transpose(1, 0, 2)

    sc_fn = pl.kernel(
        _sc_body,
        out_type=jax.ShapeDtypeStruct((4, B * COUT, VP), f32),
        mesh=plsc.VectorSubcoreMesh(
            core_axis_name="c", subcore_axis_name="s",
            num_cores=2, num_subcores=16),
        scratch_types=[
            pltpu.VMEM((NVP, 8), f32),
            pltpu.VMEM((CHUNK, 26, LANE), i32),
            pltpu.VMEM((CHUNK, 44, LANE), f32),
            pltpu.VMEM((4, 8, CHUNK * LANE), f32),
        ],
    )
    S = sc_fn(Tidx, Tw, Xr)

    A4 = coeffs.transpose(2, 0, 1)
    bias2 = jnp.broadcast_to((bias_up + bc)[:, None], (COUT, VBLK))

    out = pl.pallas_call(
        _tc_body,
        grid=(B, NVB),
        in_specs=[
            pl.BlockSpec((4, COUT, VBLK), lambda b, v: (0, b, v)),
            pl.BlockSpec((1, CIN, VBLK), lambda b, v: (b, 0, v)),
            pl.BlockSpec((4, COUT, COUT), lambda b, v: (0, 0, 0)),
            pl.BlockSpec((COUT, CIN), lambda b, v: (0, 0)),
            pl.BlockSpec((COUT, VBLK), lambda b, v: (0, 0)),
        ],
        out_specs=pl.BlockSpec((1, COUT, VBLK), lambda b, v: (b, 0, v)),
        out_shape=jax.ShapeDtypeStruct((B, COUT, V), f32),
    )(S, x2, A4, Wc, bias2)
    return out


# speedup vs baseline: 44.9905x; 44.9905x over previous
import functools

import jax
import jax.numpy as jnp
from jax import lax
from jax.experimental import pallas as pl
from jax.experimental.pallas import tpu as pltpu
from jax.experimental.pallas import tpu_sc as plsc

LEVEL = 6
V = 10 * 4 ** LEVEL + 2
F = 20 * 4 ** LEVEL
NVP = 10 * 4 ** (LEVEL - 1) + 2
B = 2
CIN = 64
COUT = 32

LANE = 16
VBLK = 512
NVB = 81
VP = NVB * VBLK
G = VP // LANE
NVR = 4
NCG = 8
GPT = G // NVR
CHUNK = 8
NCHUNK = GPT // CHUNK


def _sc_body(tidx_hbm, tw_hbm, xr_hbm, s_hbm, xpart, tidxv, twv, sbuf):
    wid = lax.axis_index("s") * 2 + lax.axis_index("c")
    cg = wid % NCG
    vr = wid // NCG

    pltpu.sync_copy(xr_hbm.at[cg], xpart)

    cvs = [jnp.full((LANE,), c, dtype=jnp.int32) for c in range(8)]

    def chunk_body(ci, carry):
        g0 = vr * GPT + ci * CHUNK
        pltpu.sync_copy(tidx_hbm.at[pl.ds(g0 * 26 * LANE, CHUNK * 26 * LANE)],
                        tidxv)
        pltpu.sync_copy(tw_hbm.at[pl.ds(g0 * 44 * LANE, CHUNK * 44 * LANE)],
                        twv)

        def group_body(gi, carry2):
            ib = gi * 26 * LANE
            wb = gi * 44 * LANE
            acc = [[jnp.zeros((LANE,), jnp.float32) for _ in range(8)]
                   for _ in range(4)]
            for j in range(26):
                idxv = tidxv[pl.ds(ib + j * LANE, LANE)]
                if j == 0:
                    w0 = twv[pl.ds(wb, LANE)]
                    for c in range(8):
                        g = plsc.load_gather(xpart, [idxv + cvs[c]])
                        acc[0][c] = acc[0][c] + w0 * g
                elif j < 8:
                    wl = twv[pl.ds(wb + j * LANE, LANE)]
                    for c in range(8):
                        g = plsc.load_gather(xpart, [idxv + cvs[c]])
                        acc[1][c] = acc[1][c] + wl * g
                else:
                    we = twv[pl.ds(wb + j * LANE, LANE)]
                    wn = twv[pl.ds(wb + (j + 18) * LANE, LANE)]
                    for c in range(8):
                        g = plsc.load_gather(xpart, [idxv + cvs[c]])
                        acc[2][c] = acc[2][c] + we * g
                        acc[3][c] = acc[3][c] + wn * g
            for k in range(4):
                for c in range(8):
                    sbuf[k, c, pl.ds(gi * LANE, LANE)] = acc[k][c]
            return carry2

        lax.fori_loop(0, CHUNK, group_body, 0, unroll=False)
        pltpu.sync_copy(
            sbuf, s_hbm.at[:, pl.ds(cg * 8, 8), pl.ds(g0 * LANE, CHUNK * LANE)])
        return carry

    lax.fori_loop(0, NCHUNK, chunk_body, 0, unroll=False)


def _tc_body(s_ref, x2_ref, a4_ref, wc_ref, bias_ref, o_ref):
    acc = bias_ref[...].astype(jnp.float32)
    for k in range(4):
        acc = acc + lax.dot_general(
            a4_ref[k], s_ref[k],
            dimension_numbers=(((1,), (0,)), ((), ())),
            preferred_element_type=jnp.float32)
    acc = acc + lax.dot_general(
        wc_ref[...], x2_ref[0],
        dimension_numbers=(((1,), (0,)), ((), ())),
        preferred_element_type=jnp.float32)
    o_ref[0] = acc


def kernel(x1, x2, coeffs, bias_up, Wc, bc, L_rows, L_cols, L_vals,
           G_rows, G_cols, G_vals, EW, NS, F2V_rows, F2V_cols, F2V_vals):
    f32 = jnp.float32
    i32 = jnp.int32

    Lc = L_cols.reshape(V, 7).astype(i32)
    Lv = L_vals.reshape(V, 7)
    fverts = G_cols.reshape(3 * F, 3)[:F].astype(i32)
    Gv_e = G_vals.reshape(3, F, 3)
    wew = jnp.einsum('fe,eft->ft', EW, Gv_e)
    wns = jnp.einsum('fe,eft->ft', NS, Gv_e)
    fv = F2V_cols.reshape(V, 6).astype(i32)
    fw = F2V_vals.reshape(V, 6)
    gidx = fverts[fv].reshape(V, 18)
    g_ew = (fw[..., None] * wew[fv]).reshape(V, 18)
    g_ns = (fw[..., None] * wns[fv]).reshape(V, 18)
    iidx = jnp.arange(V, dtype=i32)[:, None]
    iw = (iidx < NVP).astype(f32)

    IDX = jnp.concatenate([iidx, Lc, gidx], axis=1)
    ok = IDX < NVP
    IDXm = jnp.where(ok, IDX, 0) * 8
    Wt = jnp.concatenate(
        [iw * ok[:, :1], Lv * ok[:, 1:8],
         g_ew * ok[:, 8:26], g_ns * ok[:, 8:26]], axis=1)

    IDXp = jnp.zeros((VP, 26), i32).at[:V].set(IDXm)
    Wtp = jnp.zeros((VP, 44), f32).at[:V].set(Wt)
    Tidx = IDXp.reshape(G, LANE, 26).transpose(0, 2, 1).reshape(G * 26 * LANE)
    Tw = Wtp.reshape(G, LANE, 44).transpose(0, 2, 1).reshape(G * 44 * LANE)

    Xr = x1.transpose(2, 0, 1).reshape(NVP, 8, 8).transpose(1, 0, 2)
    Xr = Xr.reshape(8, NVP * 8)

    sc_fn = pl.kernel(
        _sc_body,
        out_type=jax.ShapeDtypeStruct((4, B * COUT, VP), f32),
        mesh=plsc.VectorSubcoreMesh(
            core_axis_name="c", subcore_axis_name="s",
            num_cores=2, num_subcores=16),
        compiler_params=pltpu.CompilerParams(needs_layout_passes=False),
        scratch_types=[
            pltpu.VMEM((NVP * 8,), f32),
            pltpu.VMEM((CHUNK * 26 * LANE,), i32),
            pltpu.VMEM((CHUNK * 44 * LANE,), f32),
            pltpu.VMEM((4, 8, CHUNK * LANE), f32),
        ],
    )
    S = sc_fn(Tidx, Tw, Xr)

    A4 = coeffs.transpose(2, 0, 1)
    bias2 = jnp.broadcast_to((bias_up + bc)[:, None], (COUT, VBLK))

    out = pl.pallas_call(
        _tc_body,
        grid=(B, NVB),
        in_specs=[
            pl.BlockSpec((4, COUT, VBLK), lambda b, v: (0, b, v)),
            pl.BlockSpec((1, CIN, VBLK), lambda b, v: (b, 0, v)),
            pl.BlockSpec((4, COUT, COUT), lambda b, v: (0, 0, 0)),
            pl.BlockSpec((COUT, CIN), lambda b, v: (0, 0)),
            pl.BlockSpec((COUT, VBLK), lambda b, v: (0, 0)),
        ],
        out_specs=pl.BlockSpec((1, COUT, VBLK), lambda b, v: (b, 0, v)),
        out_shape=jax.ShapeDtypeStruct((B, COUT, V), f32),
    )(S, x2, A4, Wc, bias2)
    return out


# speedup vs baseline: 48.9721x; 1.0885x over previous
import functools

import jax
import jax.numpy as jnp
from jax import lax
from jax.experimental import pallas as pl
from jax.experimental.pallas import tpu as pltpu
from jax.experimental.pallas import tpu_sc as plsc

LEVEL = 6
V = 10 * 4 ** LEVEL + 2
F = 20 * 4 ** LEVEL
NVP = 10 * 4 ** (LEVEL - 1) + 2
B = 2
CIN = 64
COUT = 32

LANE = 16
VBLK = 512
NVB = 81
VP = NVB * VBLK
G = VP // LANE
NVR = 4
NCG = 8
GPT = G // NVR
CHUNK = 8
NCHUNK = GPT // CHUNK


def _sc_body(tidx_hbm, tw_hbm, xr_hbm, s_hbm, xpart, tidxv, twv, sbuf):
    wid = lax.axis_index("s") * 2 + lax.axis_index("c")
    cg = wid % NCG
    vr = wid // NCG

    pltpu.sync_copy(xr_hbm.at[cg], xpart)

    cvs = [jnp.full((LANE,), c * NVP, dtype=jnp.int32) for c in range(8)]

    def chunk_body(ci, carry):
        g0 = vr * GPT + ci * CHUNK
        pltpu.sync_copy(tidx_hbm.at[pl.ds(g0 * 26 * LANE, CHUNK * 26 * LANE)],
                        tidxv)
        pltpu.sync_copy(tw_hbm.at[pl.ds(g0 * 44 * LANE, CHUNK * 44 * LANE)],
                        twv)

        def group_body(gi, carry2):
            ib = gi * 26 * LANE
            wb = gi * 44 * LANE
            acc = [[jnp.zeros((LANE,), jnp.float32) for _ in range(8)]
                   for _ in range(4)]
            for j in range(26):
                idxv = tidxv[pl.ds(ib + j * LANE, LANE)]
                if j == 0:
                    w0 = twv[pl.ds(wb, LANE)]
                    for c in range(8):
                        g = plsc.load_gather(xpart, [idxv + cvs[c]])
                        acc[0][c] = acc[0][c] + w0 * g
                elif j < 8:
                    wl = twv[pl.ds(wb + j * LANE, LANE)]
                    for c in range(8):
                        g = plsc.load_gather(xpart, [idxv + cvs[c]])
                        acc[1][c] = acc[1][c] + wl * g
                else:
                    we = twv[pl.ds(wb + j * LANE, LANE)]
                    wn = twv[pl.ds(wb + (j + 18) * LANE, LANE)]
                    for c in range(8):
                        g = plsc.load_gather(xpart, [idxv + cvs[c]])
                        acc[2][c] = acc[2][c] + we * g
                        acc[3][c] = acc[3][c] + wn * g
            for k in range(4):
                for c in range(8):
                    sbuf[k, c, pl.ds(gi * LANE, LANE)] = acc[k][c]
            return carry2

        lax.fori_loop(0, CHUNK, group_body, 0, unroll=False)
        pltpu.sync_copy(
            sbuf, s_hbm.at[:, pl.ds(cg * 8, 8), pl.ds(g0 * LANE, CHUNK * LANE)])
        return carry

    lax.fori_loop(0, NCHUNK, chunk_body, 0, unroll=False)


def _tc_body(s_ref, x2_ref, a4_ref, wc_ref, bias_ref, o_ref):
    acc = bias_ref[...].astype(jnp.float32)
    for k in range(4):
        acc = acc + lax.dot_general(
            a4_ref[k], s_ref[k],
            dimension_numbers=(((1,), (0,)), ((), ())),
            preferred_element_type=jnp.float32)
    acc = acc + lax.dot_general(
        wc_ref[...], x2_ref[0],
        dimension_numbers=(((1,), (0,)), ((), ())),
        preferred_element_type=jnp.float32)
    o_ref[0] = acc


def kernel(x1, x2, coeffs, bias_up, Wc, bc, L_rows, L_cols, L_vals,
           G_rows, G_cols, G_vals, EW, NS, F2V_rows, F2V_cols, F2V_vals):
    f32 = jnp.float32
    i32 = jnp.int32

    Lc = L_cols.reshape(V, 7).astype(i32)
    Lv = L_vals.reshape(V, 7)
    fverts = G_cols.reshape(3 * F, 3)[:F].astype(i32)
    Gv_e = G_vals.reshape(3, F, 3)
    wew = jnp.einsum('fe,eft->ft', EW, Gv_e)
    wns = jnp.einsum('fe,eft->ft', NS, Gv_e)
    fv = F2V_cols.reshape(V, 6).astype(i32)
    fw = F2V_vals.reshape(V, 6)
    gidx = fverts[fv].reshape(V, 18)
    g_ew = (fw[..., None] * wew[fv]).reshape(V, 18)
    g_ns = (fw[..., None] * wns[fv]).reshape(V, 18)
    iidx = jnp.arange(V, dtype=i32)[:, None]
    iw = (iidx < NVP).astype(f32)

    IDX = jnp.concatenate([iidx, Lc, gidx], axis=1)
    ok = IDX < NVP
    IDXm = jnp.where(ok, IDX, 0)
    Wt = jnp.concatenate(
        [iw * ok[:, :1], Lv * ok[:, 1:8],
         g_ew * ok[:, 8:26], g_ns * ok[:, 8:26]], axis=1)

    IDXp = jnp.zeros((VP, 26), i32).at[:V].set(IDXm)
    Wtp = jnp.zeros((VP, 44), f32).at[:V].set(Wt)
    Tidx = IDXp.reshape(G, LANE, 26).transpose(0, 2, 1).reshape(G * 26 * LANE)
    Tw = Wtp.reshape(G, LANE, 44).transpose(0, 2, 1).reshape(G * 44 * LANE)

    Xr = x1.reshape(8, 8 * NVP)

    sc_fn = pl.kernel(
        _sc_body,
        out_type=jax.ShapeDtypeStruct((4, B * COUT, VP), f32),
        mesh=plsc.VectorSubcoreMesh(
            core_axis_name="c", subcore_axis_name="s",
            num_cores=2, num_subcores=16),
        compiler_params=pltpu.CompilerParams(needs_layout_passes=False),
        scratch_types=[
            pltpu.VMEM((NVP * 8,), f32),
            pltpu.VMEM((CHUNK * 26 * LANE,), i32),
            pltpu.VMEM((CHUNK * 44 * LANE,), f32),
            pltpu.VMEM((4, 8, CHUNK * LANE), f32),
        ],
    )
    S = sc_fn(Tidx, Tw, Xr)

    A4 = coeffs.transpose(2, 0, 1)
    bias2 = jnp.broadcast_to((bias_up + bc)[:, None], (COUT, VBLK))

    out = pl.pallas_call(
        _tc_body,
        grid=(B, NVB),
        in_specs=[
            pl.BlockSpec((4, COUT, VBLK), lambda b, v: (0, b, v)),
            pl.BlockSpec((1, CIN, VBLK), lambda b, v: (b, 0, v)),
            pl.BlockSpec((4, COUT, COUT), lambda b, v: (0, 0, 0)),
            pl.BlockSpec((COUT, CIN), lambda b, v: (0, 0)),
            pl.BlockSpec((COUT, VBLK), lambda b, v: (0, 0)),
        ],
        out_specs=pl.BlockSpec((1, COUT, VBLK), lambda b, v: (b, 0, v)),
        out_shape=jax.ShapeDtypeStruct((B, COUT, V), f32),
    )(S, x2, A4, Wc, bias2)
    return out


# speedup vs baseline: 127.4997x; 2.6035x over previous
import functools

import jax
import jax.numpy as jnp
from jax import lax
from jax.experimental import pallas as pl
from jax.experimental.pallas import tpu as pltpu
from jax.experimental.pallas import tpu_sc as plsc

LEVEL = 6
V = 10 * 4 ** LEVEL + 2
F = 20 * 4 ** LEVEL
NVP = 10 * 4 ** (LEVEL - 1) + 2
B = 2
CIN = 64
COUT = 32

LANE = 16
VBLK = 512
NVB = 81
VP = NVB * VBLK

CW = 128
GPC = CW // LANE

NVR = 4
NCG = 8
NCHUNK = VP // NVR // CW

NVR_P = 3
NCHUNK_P = VP // NVR_P // CW


def _tbl_body(fld_hbm, fv_hbm, fw_hbm, lc_hbm, lv_hbm, ti_hbm, tw_hbm,
              field, fvb, fwb, lcb, lvb, stgi, stgw):
    wid = lax.axis_index("s") * 2 + lax.axis_index("c")
    role = wid % 10
    vr = wid // 10

    iota = lax.iota(jnp.int32, LANE)
    nvpv = jnp.full((LANE,), NVP, dtype=jnp.int32)
    zf = jnp.zeros((LANE,), jnp.float32)
    zi = jnp.zeros((LANE,), jnp.int32)

    @pl.when(vr < NVR_P)
    def _run():
        frow = jnp.minimum(role, 8)
        pltpu.sync_copy(fld_hbm.at[frow], field)

        def chunk_body(ci, carry):
            v0 = (vr * NCHUNK_P + ci) * CW
            pltpu.sync_copy(fv_hbm.at[:, pl.ds(v0, CW)], fvb)
            pltpu.sync_copy(fw_hbm.at[:, pl.ds(v0, CW)], fwb)
            pltpu.sync_copy(lc_hbm.at[:, pl.ds(v0, CW)], lcb)
            pltpu.sync_copy(lv_hbm.at[:, pl.ds(v0, CW)], lvb)

            @pl.when(role <= 8)
            def _field_role():
                def group_body(gi, c2):
                    l0 = gi * LANE
                    stgi[6, pl.ds(l0, LANE)] = zi
                    stgi[7, pl.ds(l0, LANE)] = zi
                    stgw[6, pl.ds(l0, LANE)] = zf
                    stgw[7, pl.ds(l0, LANE)] = zf
                    for jj in range(6):
                        fv_v = fvb[jj, pl.ds(l0, LANE)]
                        g = plsc.load_gather(field, [fv_v])
                        stgi[jj, pl.ds(l0, LANE)] = g
                        fwv = fwb[jj, pl.ds(l0, LANE)]
                        stgw[jj, pl.ds(l0, LANE)] = (
                            plsc.bitcast(g, jnp.float32) * fwv)
                    return c2

                lax.fori_loop(0, GPC, group_body, 0, unroll=False)

                @pl.when(role <= 2)
                def _vert_role():
                    pltpu.sync_copy(
                        stgi,
                        ti_hbm.at[pl.ds(8 + 8 * role, 8), pl.ds(v0, CW)])

                @pl.when(role >= 3)
                def _w_role():
                    pltpu.sync_copy(
                        stgw,
                        tw_hbm.at[pl.ds(8 + 8 * (role - 3), 8),
                                  pl.ds(v0, CW)])

            @pl.when(role == 9)
            def _idlap_role():
                def group_body(gi, c2):
                    l0 = gi * LANE
                    vids = (v0 + l0) + iota
                    ok0 = vids < nvpv
                    stgi[0, pl.ds(l0, LANE)] = jnp.where(ok0, vids, zi)
                    stgw[0, pl.ds(l0, LANE)] = ok0.astype(jnp.float32)
                    for j in range(7):
                        lcv = lcb[j, pl.ds(l0, LANE)]
                        ok = lcv < nvpv
                        stgi[1 + j, pl.ds(l0, LANE)] = jnp.where(ok, lcv, zi)
                        stgw[1 + j, pl.ds(l0, LANE)] = jnp.where(
                            ok, lvb[j, pl.ds(l0, LANE)], zf)
                    return c2

                lax.fori_loop(0, GPC, group_body, 0, unroll=False)
                pltpu.sync_copy(stgi, ti_hbm.at[pl.ds(0, 8), pl.ds(v0, CW)])
                pltpu.sync_copy(stgw, tw_hbm.at[pl.ds(0, 8), pl.ds(v0, CW)])

            return carry

        lax.fori_loop(0, NCHUNK_P, chunk_body, 0, unroll=False)


def _sc_body(ti_hbm, tw_hbm, xr_hbm, s_hbm, xpart, tib, twb, sbuf):
    wid = lax.axis_index("s") * 2 + lax.axis_index("c")
    cg = wid % NCG
    vr = wid // NCG

    pltpu.sync_copy(xr_hbm.at[cg], xpart)

    cofs = [jnp.full((LANE,), c * NVP, dtype=jnp.int32) for c in range(8)]
    nvpv = jnp.full((LANE,), NVP, dtype=jnp.int32)
    zf = jnp.zeros((LANE,), jnp.float32)
    zi = jnp.zeros((LANE,), jnp.int32)

    def chunk_body(ci, carry):
        v0 = (vr * NCHUNK + ci) * CW
        pltpu.sync_copy(ti_hbm.at[:, pl.ds(v0, CW)], tib)
        pltpu.sync_copy(tw_hbm.at[:, pl.ds(v0, CW)], twb)

        def group_body(gi, carry2):
            l0 = gi * LANE
            acc = [[zf for _ in range(8)] for _ in range(4)]
            idx0 = tib[0, pl.ds(l0, LANE)]
            w0 = twb[0, pl.ds(l0, LANE)]
            for c in range(8):
                g = plsc.load_gather(xpart, [idx0 + cofs[c]])
                acc[0][c] = w0 * g
            for j in range(1, 8):
                idx = tib[j, pl.ds(l0, LANE)]
                w = twb[j, pl.ds(l0, LANE)]
                for c in range(8):
                    g = plsc.load_gather(xpart, [idx + cofs[c]])
                    acc[1][c] = acc[1][c] + w * g
            for t in range(3):
                for jj in range(6):
                    r = 8 + 8 * t + jj
                    vert = tib[r, pl.ds(l0, LANE)]
                    ok = vert < nvpv
                    idx = jnp.where(ok, vert, zi)
                    we = jnp.where(ok, twb[r, pl.ds(l0, LANE)], zf)
                    wn = jnp.where(ok, twb[r + 24, pl.ds(l0, LANE)], zf)
                    for c in range(8):
                        g = plsc.load_gather(xpart, [idx + cofs[c]])
                        acc[2][c] = acc[2][c] + we * g
                        acc[3][c] = acc[3][c] + wn * g
            for k in range(4):
                for c in range(8):
                    sbuf[k, c, pl.ds(l0, LANE)] = acc[k][c]
            return carry2

        lax.fori_loop(0, GPC, group_body, 0, unroll=False)
        pltpu.sync_copy(sbuf, s_hbm.at[:, pl.ds(cg * 8, 8), pl.ds(v0, CW)])
        return carry

    lax.fori_loop(0, NCHUNK, chunk_body, 0, unroll=False)


def _tc_body(s_ref, x2_ref, a4_ref, wc_ref, bias_ref, o_ref):
    acc = bias_ref[...].astype(jnp.float32)
    for k in range(4):
        acc = acc + lax.dot_general(
            a4_ref[k], s_ref[k],
            dimension_numbers=(((1,), (0,)), ((), ())),
            preferred_element_type=jnp.float32)
    acc = acc + lax.dot_general(
        wc_ref[...], x2_ref[0],
        dimension_numbers=(((1,), (0,)), ((), ())),
        preferred_element_type=jnp.float32)
    o_ref[0] = acc


def kernel(x1, x2, coeffs, bias_up, Wc, bc, L_rows, L_cols, L_vals,
           G_rows, G_cols, G_vals, EW, NS, F2V_rows, F2V_cols, F2V_vals):
    f32 = jnp.float32
    i32 = jnp.int32

    LcT = jnp.zeros((7, VP), i32).at[:, :V].set(
        L_cols.reshape(V, 7).astype(i32).T)
    LvT = jnp.zeros((7, VP), f32).at[:, :V].set(L_vals.reshape(V, 7).T)
    fvT = jnp.zeros((6, VP), i32).at[:, :V].set(
        F2V_cols.reshape(V, 6).astype(i32).T)
    fwT = jnp.zeros((6, VP), f32).at[:, :V].set(F2V_vals.reshape(V, 6).T)
    fverts = G_cols.reshape(3 * F, 3)[:F].astype(i32)
    Gv_e = G_vals.reshape(3, F, 3)
    wew = jnp.einsum('fe,eft->ft', EW, Gv_e)
    wns = jnp.einsum('fe,eft->ft', NS, Gv_e)
    FLD = jnp.concatenate(
        [fverts.T,
         jax.lax.bitcast_convert_type(wew.T, i32),
         jax.lax.bitcast_convert_type(wns.T, i32)], axis=0)
    Xr = x1.reshape(8, 8 * NVP)

    tbl_fn = pl.kernel(
        _tbl_body,
        out_type=(jax.ShapeDtypeStruct((32, VP), i32),
                  jax.ShapeDtypeStruct((56, VP), f32)),
        mesh=plsc.VectorSubcoreMesh(
            core_axis_name="c", subcore_axis_name="s",
            num_cores=2, num_subcores=16),
        compiler_params=pltpu.CompilerParams(needs_layout_passes=False),
        scratch_types=[
            pltpu.VMEM((F,), i32),
            pltpu.VMEM((6, CW), i32),
            pltpu.VMEM((6, CW), f32),
            pltpu.VMEM((7, CW), i32),
            pltpu.VMEM((7, CW), f32),
            pltpu.VMEM((8, CW), i32),
            pltpu.VMEM((8, CW), f32),
        ],
    )
    TI, TW = tbl_fn(FLD, fvT, fwT, LcT, LvT)

    sc_fn = pl.kernel(
        _sc_body,
        out_type=jax.ShapeDtypeStruct((4, B * COUT, VP), f32),
        mesh=plsc.VectorSubcoreMesh(
            core_axis_name="c", subcore_axis_name="s",
            num_cores=2, num_subcores=16),
        compiler_params=pltpu.CompilerParams(needs_layout_passes=False),
        scratch_types=[
            pltpu.VMEM((8 * NVP,), f32),
            pltpu.VMEM((32, CW), i32),
            pltpu.VMEM((56, CW), f32),
            pltpu.VMEM((4, 8, CW), f32),
        ],
    )
    S = sc_fn(TI, TW, Xr)

    A4 = coeffs.transpose(2, 0, 1)
    bias2 = jnp.broadcast_to((bias_up + bc)[:, None], (COUT, VBLK))

    out = pl.pallas_call(
        _tc_body,
        grid=(B, NVB),
        in_specs=[
            pl.BlockSpec((4, COUT, VBLK), lambda b, v: (0, b, v)),
            pl.BlockSpec((1, CIN, VBLK), lambda b, v: (b, 0, v)),
            pl.BlockSpec((4, COUT, COUT), lambda b, v: (0, 0, 0)),
            pl.BlockSpec((COUT, CIN), lambda b, v: (0, 0)),
            pl.BlockSpec((COUT, VBLK), lambda b, v: (0, 0)),
        ],
        out_specs=pl.BlockSpec((1, COUT, VBLK), lambda b, v: (b, 0, v)),
        out_shape=jax.ShapeDtypeStruct((B, COUT, V), f32),
    )(S, x2, A4, Wc, bias2)
    return out


# speedup vs baseline: 173.7152x; 1.3625x over previous
import functools

import jax
import jax.numpy as jnp
from jax import lax
from jax.experimental import pallas as pl
from jax.experimental.pallas import tpu as pltpu
from jax.experimental.pallas import tpu_sc as plsc

LEVEL = 6
V = 10 * 4 ** LEVEL + 2
F = 20 * 4 ** LEVEL
NVP = 10 * 4 ** (LEVEL - 1) + 2
B = 2
CIN = 64
COUT = 32

LANE = 16
VBLK = 512
NVB = 81
VP = NVB * VBLK

CW = 128
GPC = CW // LANE
NVR = 4
NCG = 8
NCHUNK = VP // NVR // CW

CWP = 256
GPCP = CWP // LANE
NVR_P = 3
NCHUNK_P = VP // NVR_P // CWP


def _tbl_body(fld_hbm, fv_hbm, fw_hbm, lc_hbm, lv_hbm, ti_hbm, tw_hbm,
              field, fvb3, fwb3, lcb3, lvb3, stgi3, stgw3,
              semA, semB, semC, semO):
    wid = lax.axis_index("s") * 2 + lax.axis_index("c")
    role = wid % 10
    vr = wid // 10

    iota = lax.iota(jnp.int32, LANE)
    nvpv = jnp.full((LANE,), NVP, dtype=jnp.int32)
    zf = jnp.zeros((LANE,), jnp.float32)
    zi = jnp.zeros((LANE,), jnp.int32)
    nbody = NCHUNK_P // 3

    @pl.when(vr < NVR_P)
    def _run():
        frow = jnp.minimum(role, 8)
        pltpu.sync_copy(fld_hbm.at[frow], field)
        base = vr * NCHUNK_P

        def issue(ck, ph, sem):
            v0 = (base + jnp.minimum(ck, NCHUNK_P - 1)) * CWP
            return [
                pltpu.async_copy(fv_hbm.at[:, pl.ds(v0, CWP)], fvb3[ph], sem),
                pltpu.async_copy(fw_hbm.at[:, pl.ds(v0, CWP)], fwb3[ph], sem),
                pltpu.async_copy(lc_hbm.at[:, pl.ds(v0, CWP)], lcb3[ph], sem),
                pltpu.async_copy(lv_hbm.at[:, pl.ds(v0, CWP)], lvb3[ph], sem),
            ]

        def rewait(ph, sem):
            for buf, src in ((fvb3[ph], fv_hbm), (fwb3[ph], fw_hbm),
                             (lcb3[ph], lc_hbm), (lvb3[ph], lv_hbm)):
                pltpu.make_async_copy(
                    src.at[:, pl.ds(0, CWP)], buf, sem).wait()

        def compute(ck, ph):
            v0 = (base + ck) * CWP
            fvb, fwb = fvb3[ph], fwb3[ph]
            lcb, lvb = lcb3[ph], lvb3[ph]
            stgi, stgw = stgi3[ph], stgw3[ph]

            @pl.when(role <= 8)
            def _field_role():
                def group_body(gi, c2):
                    l0 = gi * LANE
                    stgi[6, pl.ds(l0, LANE)] = zi
                    stgi[7, pl.ds(l0, LANE)] = zi
                    stgw[6, pl.ds(l0, LANE)] = zf
                    stgw[7, pl.ds(l0, LANE)] = zf
                    for jj in range(6):
                        fv_v = fvb[jj, pl.ds(l0, LANE)]
                        g = plsc.load_gather(field, [fv_v])
                        stgi[jj, pl.ds(l0, LANE)] = g
                        fwv = fwb[jj, pl.ds(l0, LANE)]
                        stgw[jj, pl.ds(l0, LANE)] = (
                            plsc.bitcast(g, jnp.float32) * fwv)
                    return c2

                lax.fori_loop(0, GPCP, group_body, 0, unroll=False)

                @pl.when(role <= 2)
                def _vert_role():
                    pltpu.async_copy(
                        stgi,
                        ti_hbm.at[pl.ds(8 + 8 * role, 8), pl.ds(v0, CWP)],
                        semO)

                @pl.when(role >= 3)
                def _w_role():
                    pltpu.async_copy(
                        stgw,
                        tw_hbm.at[pl.ds(8 + 8 * (role - 3), 8),
                                  pl.ds(v0, CWP)], semO)

            @pl.when(role == 9)
            def _idlap_role():
                def group_body(gi, c2):
                    l0 = gi * LANE
                    vids = (v0 + l0) + iota
                    ok0 = vids < nvpv
                    stgi[0, pl.ds(l0, LANE)] = jnp.where(ok0, vids, zi)
                    stgw[0, pl.ds(l0, LANE)] = ok0.astype(jnp.float32)
                    for j in range(7):
                        lcv = lcb[j, pl.ds(l0, LANE)]
                        ok = lcv < nvpv
                        stgi[1 + j, pl.ds(l0, LANE)] = jnp.where(ok, lcv, zi)
                        stgw[1 + j, pl.ds(l0, LANE)] = jnp.where(
                            ok, lvb[j, pl.ds(l0, LANE)], zf)
                    return c2

                lax.fori_loop(0, GPCP, group_body, 0, unroll=False)
                pltpu.async_copy(
                    stgi, ti_hbm.at[pl.ds(0, 8), pl.ds(v0, CWP)], semO)
                pltpu.async_copy(
                    stgw, tw_hbm.at[pl.ds(0, 8), pl.ds(v0, CWP)], semO)

        def drain_out():
            for _ in range(3):
                pltpu.make_async_copy(
                    ti_hbm.at[pl.ds(0, 8), pl.ds(0, CWP)], stgi3[0],
                    semO).wait()

            @pl.when(role == 9)
            def _extra():
                for _ in range(3):
                    pltpu.make_async_copy(
                        ti_hbm.at[pl.ds(0, 8), pl.ds(0, CWP)], stgi3[0],
                        semO).wait()

        hA = issue(0, 0, semA)
        issue(1, 1, semB)
        for cp in hA:
            cp.wait()

        def body(k, carry):
            ck = k * 3

            @pl.when(k > 0)
            def _drain():
                drain_out()

            hC = issue(ck + 2, 2, semC)
            compute(ck, 0)
            rewait(1, semB)
            hA2 = issue(ck + 3, 0, semA)
            compute(ck + 1, 1)
            for cp in hC:
                cp.wait()
            issue(ck + 4, 1, semB)
            compute(ck + 2, 2)
            for cp in hA2:
                cp.wait()
            return carry

        lax.fori_loop(0, nbody, body, 0, unroll=False)
        drain_out()
        rewait(1, semB)


def _sc_body(ti_hbm, tw_hbm, xr_hbm, s_hbm,
             xpart, tib3, twb3, sbuf3, semA, semB, semC, semO):
    wid = lax.axis_index("s") * 2 + lax.axis_index("c")
    cg = wid % NCG
    vr = wid // NCG

    pltpu.sync_copy(xr_hbm.at[cg], xpart)

    cofs = [jnp.full((LANE,), c * NVP, dtype=jnp.int32) for c in range(8)]
    nvpv = jnp.full((LANE,), NVP, dtype=jnp.int32)
    zf = jnp.zeros((LANE,), jnp.float32)
    zi = jnp.zeros((LANE,), jnp.int32)
    base = vr * NCHUNK
    nbody = NCHUNK // 3

    def issue(ck, ph, sem):
        v0 = (base + jnp.minimum(ck, NCHUNK - 1)) * CW
        return [
            pltpu.async_copy(ti_hbm.at[:, pl.ds(v0, CW)], tib3[ph], sem),
            pltpu.async_copy(tw_hbm.at[:, pl.ds(v0, CW)], twb3[ph], sem),
        ]

    def rewait(ph, sem):
        pltpu.make_async_copy(
            ti_hbm.at[:, pl.ds(0, CW)], tib3[ph], sem).wait()
        pltpu.make_async_copy(
            tw_hbm.at[:, pl.ds(0, CW)], twb3[ph], sem).wait()

    def compute(ck, ph):
        v0 = (base + ck) * CW
        tib, twb, sbuf = tib3[ph], twb3[ph], sbuf3[ph]

        def group_body(gi, carry2):
            l0 = gi * LANE
            acc = [[zf for _ in range(8)] for _ in range(4)]
            idx0 = tib[0, pl.ds(l0, LANE)]
            w0 = twb[0, pl.ds(l0, LANE)]
            for c in range(8):
                g = plsc.load_gather(xpart, [idx0 + cofs[c]])
                acc[0][c] = w0 * g
            for j in range(1, 8):
                idx = tib[j, pl.ds(l0, LANE)]
                w = twb[j, pl.ds(l0, LANE)]
                for c in range(8):
                    g = plsc.load_gather(xpart, [idx + cofs[c]])
                    acc[1][c] = acc[1][c] + w * g
            for t in range(3):
                for jj in range(6):
                    r = 8 + 8 * t + jj
                    vert = tib[r, pl.ds(l0, LANE)]
                    ok = vert < nvpv
                    idx = jnp.where(ok, vert, zi)
                    we = jnp.where(ok, twb[r, pl.ds(l0, LANE)], zf)
                    wn = jnp.where(ok, twb[r + 24, pl.ds(l0, LANE)], zf)
                    for c in range(8):
                        g = plsc.load_gather(xpart, [idx + cofs[c]])
                        acc[2][c] = acc[2][c] + we * g
                        acc[3][c] = acc[3][c] + wn * g
            for k in range(4):
                for c in range(8):
                    sbuf[k, c, pl.ds(l0, LANE)] = acc[k][c]
            return carry2

        lax.fori_loop(0, GPC, group_body, 0, unroll=False)
        pltpu.async_copy(
            sbuf, s_hbm.at[:, pl.ds(cg * 8, 8), pl.ds(v0, CW)], semO)

    def drain_out():
        for _ in range(3):
            pltpu.make_async_copy(
                s_hbm.at[:, pl.ds(0, 8), pl.ds(0, CW)], sbuf3[0],
                semO).wait()

    hA = issue(0, 0, semA)
    issue(1, 1, semB)
    for cp in hA:
        cp.wait()

    def body(k, carry):
        ck = k * 3

        @pl.when(k > 0)
        def _drain():
            drain_out()

        hC = issue(ck + 2, 2, semC)
        compute(ck, 0)
        rewait(1, semB)
        hA2 = issue(ck + 3, 0, semA)
        compute(ck + 1, 1)
        for cp in hC:
            cp.wait()
        issue(ck + 4, 1, semB)
        compute(ck + 2, 2)
        for cp in hA2:
            cp.wait()
        return carry

    lax.fori_loop(0, nbody, body, 0, unroll=False)
    drain_out()
    rewait(1, semB)


def _tc_body(s_ref, x2_ref, a4_ref, wc_ref, bias_ref, o_ref):
    acc = bias_ref[...].astype(jnp.float32)
    for k in range(4):
        acc = acc + lax.dot_general(
            a4_ref[k], s_ref[k],
            dimension_numbers=(((1,), (0,)), ((), ())),
            preferred_element_type=jnp.float32)
    acc = acc + lax.dot_general(
        wc_ref[...], x2_ref[0],
        dimension_numbers=(((1,), (0,)), ((), ())),
        preferred_element_type=jnp.float32)
    o_ref[0] = acc


def kernel(x1, x2, coeffs, bias_up, Wc, bc, L_rows, L_cols, L_vals,
           G_rows, G_cols, G_vals, EW, NS, F2V_rows, F2V_cols, F2V_vals):
    f32 = jnp.float32
    i32 = jnp.int32

    LcT = jnp.zeros((7, VP), i32).at[:, :V].set(
        L_cols.reshape(V, 7).astype(i32).T)
    LvT = jnp.zeros((7, VP), f32).at[:, :V].set(L_vals.reshape(V, 7).T)
    fvT = jnp.zeros((6, VP), i32).at[:, :V].set(
        F2V_cols.reshape(V, 6).astype(i32).T)
    fwT = jnp.zeros((6, VP), f32).at[:, :V].set(F2V_vals.reshape(V, 6).T)
    fverts = G_cols.reshape(3 * F, 3)[:F].astype(i32)
    Gv_e = G_vals.reshape(3, F, 3)
    wew = jnp.einsum('fe,eft->ft', EW, Gv_e)
    wns = jnp.einsum('fe,eft->ft', NS, Gv_e)
    FLD = jnp.concatenate(
        [fverts.T,
         jax.lax.bitcast_convert_type(wew.T, i32),
         jax.lax.bitcast_convert_type(wns.T, i32)], axis=0)
    Xr = x1.reshape(8, 8 * NVP)

    tbl_fn = pl.kernel(
        _tbl_body,
        out_type=(jax.ShapeDtypeStruct((32, VP), i32),
                  jax.ShapeDtypeStruct((56, VP), f32)),
        mesh=plsc.VectorSubcoreMesh(
            core_axis_name="c", subcore_axis_name="s",
            num_cores=2, num_subcores=16),
        compiler_params=pltpu.CompilerParams(needs_layout_passes=False),
        scratch_types=[
            pltpu.VMEM((F,), i32),
            [pltpu.VMEM((6, CWP), i32) for _ in range(3)],
            [pltpu.VMEM((6, CWP), f32) for _ in range(3)],
            [pltpu.VMEM((7, CWP), i32) for _ in range(3)],
            [pltpu.VMEM((7, CWP), f32) for _ in range(3)],
            [pltpu.VMEM((8, CWP), i32) for _ in range(3)],
            [pltpu.VMEM((8, CWP), f32) for _ in range(3)],
            pltpu.SemaphoreType.DMA,
            pltpu.SemaphoreType.DMA,
            pltpu.SemaphoreType.DMA,
            pltpu.SemaphoreType.DMA,
        ],
    )
    TI, TW = tbl_fn(FLD, fvT, fwT, LcT, LvT)

    sc_fn = pl.kernel(
        _sc_body,
        out_type=jax.ShapeDtypeStruct((4, B * COUT, VP), f32),
        mesh=plsc.VectorSubcoreMesh(
            core_axis_name="c", subcore_axis_name="s",
            num_cores=2, num_subcores=16),
        compiler_params=pltpu.CompilerParams(needs_layout_passes=False),
        scratch_types=[
            pltpu.VMEM((8 * NVP,), f32),
            [pltpu.VMEM((32, CW), i32) for _ in range(3)],
            [pltpu.VMEM((56, CW), f32) for _ in range(3)],
            [pltpu.VMEM((4, 8, CW), f32) for _ in range(3)],
            pltpu.SemaphoreType.DMA,
            pltpu.SemaphoreType.DMA,
            pltpu.SemaphoreType.DMA,
            pltpu.SemaphoreType.DMA,
        ],
    )
    S = sc_fn(TI, TW, Xr)

    A4 = coeffs.transpose(2, 0, 1)
    bias2 = jnp.broadcast_to((bias_up + bc)[:, None], (COUT, VBLK))

    out = pl.pallas_call(
        _tc_body,
        grid=(B, NVB),
        in_specs=[
            pl.BlockSpec((4, COUT, VBLK), lambda b, v: (0, b, v)),
            pl.BlockSpec((1, CIN, VBLK), lambda b, v: (b, 0, v)),
            pl.BlockSpec((4, COUT, COUT), lambda b, v: (0, 0, 0)),
            pl.BlockSpec((COUT, CIN), lambda b, v: (0, 0)),
            pl.BlockSpec((COUT, VBLK), lambda b, v: (0, 0)),
        ],
        out_specs=pl.BlockSpec((1, COUT, VBLK), lambda b, v: (b, 0, v)),
        out_shape=jax.ShapeDtypeStruct((B, COUT, V), f32),
    )(S, x2, A4, Wc, bias2)
    return out
